# Initial kernel scaffold; baseline (speedup 1.0000x reference)
import functools

import jax
import jax.numpy as jnp
from jax import lax
from jax.experimental import pallas as pl
from jax.experimental.pallas import tpu as pltpu
from jax.experimental.pallas import tpu_sc as plsc

VOCAB = 1000000
HIDDEN = 64
BATCH = 16384
NEG = 5

NUM_CORES = 2
NUM_SUBCORES = 16
NW = NUM_CORES * NUM_SUBCORES
BPW = BATCH // NW
NGATHER = 2 + NEG

_sc_mesh = plsc.VectorSubcoreMesh(core_axis_name="c", subcore_axis_name="s")


@functools.partial(
    pl.kernel,
    out_type=[
        jax.ShapeDtypeStruct((BATCH, HIDDEN), jnp.float32),
        jax.ShapeDtypeStruct((BATCH, HIDDEN), jnp.float32),
        jax.ShapeDtypeStruct((NEG * BATCH, HIDDEN), jnp.float32),
    ],
    mesh=_sc_mesh,
    scratch_types=[
        pltpu.VMEM((NGATHER * BPW,), jnp.int32),
        pltpu.VMEM((BPW, HIDDEN), jnp.float32),
        pltpu.VMEM((BPW, HIDDEN), jnp.float32),
        pltpu.SemaphoreType.DMA,
        pltpu.SemaphoreType.DMA,
    ],
)
def _sc_gather(syn0, syn1, idx_all, out0, out1, outs,
               idx_v, rows_a, rows_b, sem_a, sem_b):
    wid = lax.axis_index("s") * NUM_CORES + lax.axis_index("c")
    base = wid * BPW

    for j in range(NGATHER):
        pltpu.sync_copy(idx_all.at[pl.ds(j * BATCH + base, BPW)],
                        idx_v.at[pl.ds(j * BPW, BPW)])

    bufs = (rows_a, rows_b)
    sems = (sem_a, sem_b)
    copies = [None, None]

    def _table(j):
        return syn0 if j == 0 else syn1

    def _dst(j):
        if j == 0:
            return out0.at[pl.ds(base, BPW)]
        if j == 1:
            return out1.at[pl.ds(base, BPW)]
        return outs.at[pl.ds((j - 2) * BATCH + base, BPW)]

    for j in range(NGATHER):
        s = j % 2
        copies[s] = pltpu.async_copy(
            _table(j).at[idx_v.at[pl.ds(j * BPW, BPW)]], bufs[s], sems[s])
        if j > 0:
            copies[1 - s].wait()
            pltpu.sync_copy(bufs[1 - s], _dst(j - 1))
    copies[(NGATHER - 1) % 2].wait()
    pltpu.sync_copy(bufs[(NGATHER - 1) % 2], _dst(NGATHER - 1))


_TC_BLK = 2048


def _tc_loss_body(in0_ref, t1_ref, s1_ref, out_ref):
    a = in0_ref[...]
    tl = jnp.sum(a * t1_ref[...], axis=1)
    out_ref[0, :] = jax.nn.softplus(-tl)
    for n in range(NEG):
        sl = jnp.sum(a * s1_ref[n], axis=1)
        out_ref[1 + n, :] = jax.nn.softplus(sl)
    out_ref[6, :] = jnp.zeros((_TC_BLK,), jnp.float32)
    out_ref[7, :] = jnp.zeros((_TC_BLK,), jnp.float32)


_tc_loss = pl.pallas_call(
    _tc_loss_body,
    grid=(BATCH // _TC_BLK,),
    in_specs=[
        pl.BlockSpec((_TC_BLK, HIDDEN), lambda i: (i, 0)),
        pl.BlockSpec((_TC_BLK, HIDDEN), lambda i: (i, 0)),
        pl.BlockSpec((NEG, _TC_BLK, HIDDEN), lambda i: (0, i, 0)),
    ],
    out_specs=pl.BlockSpec((8, _TC_BLK), lambda i: (0, i)),
    out_shape=jax.ShapeDtypeStruct((8, BATCH), jnp.float32),
)


def kernel(syn0, syn1, inputs, labels, sampled_ids):
    idx_all = jnp.concatenate([
        inputs.astype(jnp.int32),
        labels.astype(jnp.int32),
        sampled_ids.astype(jnp.int32).reshape(-1),
    ])
    in0, t1, s1 = _sc_gather(syn0, syn1, idx_all)
    loss_t = _tc_loss(in0, t1, s1.reshape(NEG, BATCH, HIDDEN))
    return loss_t[:6].T


# speedup vs baseline: 1.5569x; 1.5569x over previous
import functools

import jax
import jax.numpy as jnp
from jax import lax
from jax.experimental import pallas as pl
from jax.experimental.pallas import tpu as pltpu
from jax.experimental.pallas import tpu_sc as plsc

VOCAB = 1000000
HIDDEN = 64
BATCH = 16384
NEG = 5

NUM_CORES = 2
NUM_SUBCORES = 16
NW = NUM_CORES * NUM_SUBCORES
BPW = BATCH // NW
NGATHER = 2 + NEG

_sc_mesh = plsc.VectorSubcoreMesh(core_axis_name="c", subcore_axis_name="s")


@functools.partial(
    pl.kernel,
    out_type=[
        jax.ShapeDtypeStruct((BATCH, HIDDEN), jnp.float32),
        jax.ShapeDtypeStruct((BATCH, HIDDEN), jnp.float32),
        jax.ShapeDtypeStruct((NEG * BATCH, HIDDEN), jnp.float32),
    ],
    mesh=_sc_mesh,
    compiler_params=pltpu.CompilerParams(use_tc_tiling_on_sc=False),
    scratch_types=[
        pltpu.VMEM((NGATHER * BPW,), jnp.int32),
        pltpu.VMEM((BPW, HIDDEN), jnp.float32),
        pltpu.VMEM((BPW, HIDDEN), jnp.float32),
        pltpu.SemaphoreType.DMA,
        pltpu.SemaphoreType.DMA,
    ],
)
def _sc_gather(syn0, syn1, idx_all, out0, out1, outs,
               idx_v, rows_a, rows_b, sem_a, sem_b):
    wid = lax.axis_index("s") * NUM_CORES + lax.axis_index("c")
    base = wid * BPW

    for j in range(NGATHER):
        pltpu.sync_copy(idx_all.at[pl.ds(j * BATCH + base, BPW)],
                        idx_v.at[pl.ds(j * BPW, BPW)])

    bufs = (rows_a, rows_b)
    sems = (sem_a, sem_b)
    copies = [None, None]

    def _table(j):
        return syn0 if j == 0 else syn1

    def _dst(j):
        if j == 0:
            return out0.at[pl.ds(base, BPW)]
        if j == 1:
            return out1.at[pl.ds(base, BPW)]
        return outs.at[pl.ds((j - 2) * BATCH + base, BPW)]

    for j in range(NGATHER):
        s = j % 2
        copies[s] = pltpu.async_copy(
            _table(j).at[idx_v.at[pl.ds(j * BPW, BPW)]], bufs[s], sems[s])
        if j > 0:
            copies[1 - s].wait()
            pltpu.sync_copy(bufs[1 - s], _dst(j - 1))
    copies[(NGATHER - 1) % 2].wait()
    pltpu.sync_copy(bufs[(NGATHER - 1) % 2], _dst(NGATHER - 1))


_TC_BLK = 2048


def _tc_loss_body(in0_ref, t1_ref, s1_ref, out_ref):
    a = in0_ref[...]
    tl = jnp.sum(a * t1_ref[...], axis=1)
    out_ref[0, :] = jax.nn.softplus(-tl)
    for n in range(NEG):
        sl = jnp.sum(a * s1_ref[n], axis=1)
        out_ref[1 + n, :] = jax.nn.softplus(sl)
    out_ref[6, :] = jnp.zeros((_TC_BLK,), jnp.float32)
    out_ref[7, :] = jnp.zeros((_TC_BLK,), jnp.float32)


_tc_loss = pl.pallas_call(
    _tc_loss_body,
    grid=(BATCH // _TC_BLK,),
    in_specs=[
        pl.BlockSpec((_TC_BLK, HIDDEN), lambda i: (i, 0)),
        pl.BlockSpec((_TC_BLK, HIDDEN), lambda i: (i, 0)),
        pl.BlockSpec((NEG, _TC_BLK, HIDDEN), lambda i: (0, i, 0)),
    ],
    out_specs=pl.BlockSpec((8, _TC_BLK), lambda i: (0, i)),
    out_shape=jax.ShapeDtypeStruct((8, BATCH), jnp.float32),
)


def kernel(syn0, syn1, inputs, labels, sampled_ids):
    idx_all = jnp.concatenate([
        inputs.astype(jnp.int32),
        labels.astype(jnp.int32),
        sampled_ids.astype(jnp.int32).reshape(-1),
    ])
    in0, t1, s1 = _sc_gather(syn0, syn1, idx_all)
    loss_t = _tc_loss(in0, t1, s1.reshape(NEG, BATCH, HIDDEN))
    return loss_t[:6].T
